# baseline (speedup 1.0000x reference)
import functools

import jax
import jax.numpy as jnp
import numpy as np
from jax import lax
from jax.experimental import pallas as pl
from jax.experimental.pallas import tpu as pltpu
from jax.experimental.pallas import tpu_sc as plsc

S0 = 8192
F = 4096
RANK = 2048
DROP_RATE = 0.5

NC, NS, L = 2, 16, 16
NW = NC * NS
ROWS_PER_W = S0 // NW
BATCH_ROWS = L
NBATCH = ROWS_PER_W // BATCH_ROWS
MASK_WORDS = F // 32

_ROW_BLOCK = 128


def _threefry2x32_np(k0, k1, x0, x1):
    x0 = x0.astype(np.uint32).copy()
    x1 = x1.astype(np.uint32).copy()
    ks0 = np.uint32(k0)
    ks1 = np.uint32(k1)
    ks2 = np.uint32(np.uint32(0x1BD11BDA) ^ ks0 ^ ks1)
    ks = [ks0, ks1, ks2]
    rotations = [13, 15, 26, 6, 17, 29, 16, 24]
    x0 += ks0
    x1 += ks1
    for d in range(20):
        r = rotations[d % 8]
        x0 += x1
        x1 = (x1 << np.uint32(r)) | (x1 >> np.uint32(32 - r))
        x1 ^= x0
        if (d + 1) % 4 == 0:
            j = (d + 1) // 4
            x0 += ks[j % 3]
            x1 += ks[(j + 1) % 3] + np.uint32(j)
    return x0, x1


@functools.cache
def _drop_mask_np():
    n = F * S0
    bits = np.empty(n, dtype=np.uint32)
    chunk = 1 << 22
    for start in range(0, n, chunk):
        c = np.arange(start, min(start + chunk, n), dtype=np.uint32)
        o0, o1 = _threefry2x32_np(0, 1234, np.zeros_like(c), c)
        bits[start:start + c.size] = o0 ^ o1
    fb = (bits >> np.uint32(9)) | np.uint32(0x3F800000)
    f = fb.view(np.float32) - np.float32(1.0)
    u = f * np.float32(0.9) + np.float32(0.1)
    u = np.maximum(np.float32(0.1), u)
    return np.ascontiguousarray((u <= np.float32(DROP_RATE)).reshape(F, S0).T)


@functools.cache
def _drop_mask_words_np():
    m = _drop_mask_np().reshape(S0, MASK_WORDS, 32).astype(np.uint64)
    w = (m << np.arange(32, dtype=np.uint64)).sum(axis=2)
    return np.ascontiguousarray(w.astype(np.uint32).view(np.int32))


def _sc_body(x_hbm, mw_hbm, o_hbm, data, hist, mw, *, rows_per_w, nbatch):
    cid = lax.axis_index("c")
    sid = lax.axis_index("s")
    wid = sid * NC + cid
    row0 = wid * rows_per_w
    iota = lax.iota(jnp.int32, L)
    rbase = iota * F
    mbase = iota * MASK_WORDS
    ones = jnp.ones((L,), jnp.int32)
    zeros16 = jnp.zeros((L,), jnp.int32)

    @plsc.parallel_loop(0, 2048, unroll=8)
    def zb(b):
        hist[pl.ds(b * L, L)] = zeros16

    def hist_pass(bucket_and_mask):
        @plsc.parallel_loop(0, F, unroll=8)
        def hb(p):
            v = plsc.load_gather(data, [rbase + p])
            bits = plsc.bitcast(v, jnp.int32) & jnp.int32(0x7FFFFFFF)
            b, m = bucket_and_mask(bits)
            plsc.addupdate_scatter(hist, [b * L + iota], ones, mask=m)

    def scan_pass(nb, kk):
        def sb(b, carry):
            c, bsel, csel = carry
            h = hist[pl.ds(b * L, L)]
            hist[pl.ds(b * L, L)] = zeros16
            nc = c + h
            sel = (c <= kk) & (nc > kk)
            bsel = jnp.where(sel, jnp.full((L,), b, jnp.int32), bsel)
            csel = jnp.where(sel, c, csel)
            return (nc, bsel, csel)
        _, bsel, csel = lax.fori_loop(0, nb, sb, (zeros16, zeros16, zeros16))
        return bsel, kk - csel

    def batch_body(bi, carry):
        r0 = row0 + bi * BATCH_ROWS
        pltpu.sync_copy(x_hbm.at[pl.ds(r0 * F, BATCH_ROWS * F)], data)
        pltpu.sync_copy(mw_hbm.at[pl.ds(r0 * MASK_WORDS, BATCH_ROWS * MASK_WORDS)], mw)

        hist_pass(lambda bits: (lax.shift_right_logical(bits, 21), None))
        kk = jnp.full((L,), RANK, jnp.int32)
        p1, kk = scan_pass(1024, kk)

        def bm2(bits):
            b = lax.shift_right_logical(bits, 11) & jnp.int32(1023)
            m = lax.shift_right_logical(bits, 21) == p1
            return b, m
        hist_pass(bm2)
        p2, kk = scan_pass(1024, kk)

        pref12 = (p1 << 10) | p2
        def bm3(bits):
            b = bits & jnp.int32(2047)
            m = lax.shift_right_logical(bits, 11) == pref12
            return b, m
        hist_pass(bm3)
        p3, _ = scan_pass(2048, kk)

        tbits = (p1 << 21) | (p2 << 11) | p3
        thr = plsc.bitcast(tbits, jnp.float32)

        @plsc.parallel_loop(0, MASK_WORDS, unroll=2)
        def mb(wi):
            w = plsc.load_gather(mw, [mbase + wi])
            base = rbase + wi * 32
            for sub in range(32):
                idx = base + sub
                v = plsc.load_gather(data, [idx])
                a = jnp.abs(v)
                bit = lax.shift_right_logical(w, jnp.int32(sub)) & jnp.int32(1)
                drop = (a <= thr) & (bit == 1)
                res = jnp.where(drop, jnp.float32(0.0), a)
                plsc.store_scatter(data, [idx], res)

        pltpu.sync_copy(data, o_hbm.at[pl.ds(r0 * F, BATCH_ROWS * F)])
        return carry

    lax.fori_loop(0, nbatch, batch_body, 0)


def _sc_kernel(x, mask_words):
    nrows = x.shape[0]
    rows_per_w = nrows // NW
    nbatch = rows_per_w // BATCH_ROWS
    mesh = plsc.VectorSubcoreMesh(core_axis_name="c", subcore_axis_name="s")
    k = pl.kernel(
        functools.partial(_sc_body, rows_per_w=rows_per_w, nbatch=nbatch),
        out_type=jax.ShapeDtypeStruct((nrows * F,), jnp.float32),
        mesh=mesh,
        scratch_types=[
            pltpu.VMEM((BATCH_ROWS * F,), jnp.float32),
            pltpu.VMEM((2048 * L,), jnp.int32),
            pltpu.VMEM((BATCH_ROWS * MASK_WORDS,), jnp.int32),
        ],
        compiler_params=pltpu.CompilerParams(needs_layout_passes=False),
    )
    flat = k(x.reshape(nrows * F), mask_words.reshape(nrows * MASK_WORDS))
    return flat.reshape(nrows, F)


def _count_lt(arr, cand, min_width):
    s = jnp.where(arr < cand, jnp.int16(1), jnp.int16(0))
    n = s.shape[1]
    while n > min_width:
        h = n // 2
        s = s[:, :h] + s[:, h:]
        n = h
    return jnp.sum(s.astype(jnp.int32), axis=1, keepdims=True)


def _tc16_body(x_ref, m_ref, o_ref, *, rank):
    x = x_ref[...]
    a = jnp.abs(x)
    bits = jax.lax.bitcast_convert_type(a, jnp.int32)
    rows = x.shape[0]
    hi = lax.shift_right_logical(bits, 16).astype(jnp.int16)

    def step1(_, carry):
        t, bit = carry
        cand = t | bit
        cnt = _count_lt(hi, cand.astype(jnp.int16), 256)
        t = jnp.where(cnt <= rank, cand, t)
        return t, bit >> 1

    t0 = jnp.zeros((rows, 1), dtype=jnp.int32)
    hi_t, _ = jax.lax.fori_loop(0, 15, step1, (t0, jnp.int32(1 << 14)))
    hi_t16 = hi_t.astype(jnp.int16)

    lo_s = ((bits & jnp.int32(0xFFFF)) - jnp.int32(32768)).astype(jnp.int16)
    below = hi < hi_t16
    eq = hi == hi_t16
    s2 = jnp.where(eq, lo_s, jnp.where(below, jnp.int16(-32768), jnp.int16(32767)))

    def step2(_, carry):
        t, bit = carry
        cand = t | bit
        cnt = _count_lt(s2, (cand - jnp.int32(32768)).astype(jnp.int16), 256)
        t = jnp.where(cnt <= rank, cand, t)
        return t, bit >> 1

    lo_t, _ = jax.lax.fori_loop(0, 16, step2, (t0, jnp.int32(1 << 15)))

    t_bits = (hi_t << 16) | lo_t
    thr = jax.lax.bitcast_convert_type(t_bits, jnp.float32)
    drop = (a * m_ref[...]) <= thr
    o_ref[...] = jnp.where(drop, jnp.zeros_like(a), a)


def _tc16_kernel(x, mask_scale_f32):
    nrows = x.shape[0]
    grid = (nrows // _ROW_BLOCK,)
    return pl.pallas_call(
        functools.partial(_tc16_body, rank=RANK),
        grid=grid,
        in_specs=[
            pl.BlockSpec((_ROW_BLOCK, F), lambda i: (i, 0)),
            pl.BlockSpec((_ROW_BLOCK, F), lambda i: (i, 0)),
        ],
        out_specs=pl.BlockSpec((_ROW_BLOCK, F), lambda i: (i, 0)),
        out_shape=jax.ShapeDtypeStruct((nrows, F), jnp.float32),
    )(x, mask_scale_f32)


@functools.cache
def _mask_scale_np():
    return np.where(_drop_mask_np(), np.float32(1.0),
                    np.float32(np.inf)).astype(np.float32)


def kernel(input):
    return _tc16_kernel(input, jnp.asarray(_mask_scale_np()))


# speedup vs baseline: 1.1968x; 1.1968x over previous
import functools

import jax
import jax.numpy as jnp
import numpy as np
from jax import lax
from jax.experimental import pallas as pl
from jax.experimental.pallas import tpu as pltpu
from jax.experimental.pallas import tpu_sc as plsc

S0 = 8192
F = 4096
RANK = 2048
DROP_RATE = 0.5

NC, NS, L = 2, 16, 16
NW = NC * NS
ROWS_PER_W = S0 // NW
BATCH_ROWS = L
NBATCH = ROWS_PER_W // BATCH_ROWS
MASK_WORDS = F // 32

_ROW_BLOCK = 256


def _threefry2x32_np(k0, k1, x0, x1):
    x0 = x0.astype(np.uint32).copy()
    x1 = x1.astype(np.uint32).copy()
    ks0 = np.uint32(k0)
    ks1 = np.uint32(k1)
    ks2 = np.uint32(np.uint32(0x1BD11BDA) ^ ks0 ^ ks1)
    ks = [ks0, ks1, ks2]
    rotations = [13, 15, 26, 6, 17, 29, 16, 24]
    x0 += ks0
    x1 += ks1
    for d in range(20):
        r = rotations[d % 8]
        x0 += x1
        x1 = (x1 << np.uint32(r)) | (x1 >> np.uint32(32 - r))
        x1 ^= x0
        if (d + 1) % 4 == 0:
            j = (d + 1) // 4
            x0 += ks[j % 3]
            x1 += ks[(j + 1) % 3] + np.uint32(j)
    return x0, x1


@functools.cache
def _drop_mask_np():
    n = F * S0
    bits = np.empty(n, dtype=np.uint32)
    chunk = 1 << 22
    for start in range(0, n, chunk):
        c = np.arange(start, min(start + chunk, n), dtype=np.uint32)
        o0, o1 = _threefry2x32_np(0, 1234, np.zeros_like(c), c)
        bits[start:start + c.size] = o0 ^ o1
    fb = (bits >> np.uint32(9)) | np.uint32(0x3F800000)
    f = fb.view(np.float32) - np.float32(1.0)
    u = f * np.float32(0.9) + np.float32(0.1)
    u = np.maximum(np.float32(0.1), u)
    return np.ascontiguousarray((u <= np.float32(DROP_RATE)).reshape(F, S0).T)


@functools.cache
def _drop_mask_words_np():
    m = _drop_mask_np().reshape(S0, MASK_WORDS, 32).astype(np.uint64)
    w = (m << np.arange(32, dtype=np.uint64)).sum(axis=2)
    return np.ascontiguousarray(w.astype(np.uint32).view(np.int32))


def _sc_body(x_hbm, mw_hbm, o_hbm, data, hist, mw, *, rows_per_w, nbatch):
    cid = lax.axis_index("c")
    sid = lax.axis_index("s")
    wid = sid * NC + cid
    row0 = wid * rows_per_w
    iota = lax.iota(jnp.int32, L)
    rbase = iota * F
    mbase = iota * MASK_WORDS
    ones = jnp.ones((L,), jnp.int32)
    zeros16 = jnp.zeros((L,), jnp.int32)

    @plsc.parallel_loop(0, 2048, unroll=8)
    def zb(b):
        hist[pl.ds(b * L, L)] = zeros16

    def hist_pass(bucket_and_mask):
        @plsc.parallel_loop(0, F, unroll=8)
        def hb(p):
            v = plsc.load_gather(data, [rbase + p])
            bits = plsc.bitcast(v, jnp.int32) & jnp.int32(0x7FFFFFFF)
            b, m = bucket_and_mask(bits)
            plsc.addupdate_scatter(hist, [b * L + iota], ones, mask=m)

    def scan_pass(nb, kk):
        def sb(b, carry):
            c, bsel, csel = carry
            h = hist[pl.ds(b * L, L)]
            hist[pl.ds(b * L, L)] = zeros16
            nc = c + h
            sel = (c <= kk) & (nc > kk)
            bsel = jnp.where(sel, jnp.full((L,), b, jnp.int32), bsel)
            csel = jnp.where(sel, c, csel)
            return (nc, bsel, csel)
        _, bsel, csel = lax.fori_loop(0, nb, sb, (zeros16, zeros16, zeros16))
        return bsel, kk - csel

    def batch_body(bi, carry):
        r0 = row0 + bi * BATCH_ROWS
        pltpu.sync_copy(x_hbm.at[pl.ds(r0 * F, BATCH_ROWS * F)], data)
        pltpu.sync_copy(mw_hbm.at[pl.ds(r0 * MASK_WORDS, BATCH_ROWS * MASK_WORDS)], mw)

        hist_pass(lambda bits: (lax.shift_right_logical(bits, 21), None))
        kk = jnp.full((L,), RANK, jnp.int32)
        p1, kk = scan_pass(1024, kk)

        def bm2(bits):
            b = lax.shift_right_logical(bits, 11) & jnp.int32(1023)
            m = lax.shift_right_logical(bits, 21) == p1
            return b, m
        hist_pass(bm2)
        p2, kk = scan_pass(1024, kk)

        pref12 = (p1 << 10) | p2
        def bm3(bits):
            b = bits & jnp.int32(2047)
            m = lax.shift_right_logical(bits, 11) == pref12
            return b, m
        hist_pass(bm3)
        p3, _ = scan_pass(2048, kk)

        tbits = (p1 << 21) | (p2 << 11) | p3
        thr = plsc.bitcast(tbits, jnp.float32)

        @plsc.parallel_loop(0, MASK_WORDS, unroll=2)
        def mb(wi):
            w = plsc.load_gather(mw, [mbase + wi])
            base = rbase + wi * 32
            for sub in range(32):
                idx = base + sub
                v = plsc.load_gather(data, [idx])
                a = jnp.abs(v)
                bit = lax.shift_right_logical(w, jnp.int32(sub)) & jnp.int32(1)
                drop = (a <= thr) & (bit == 1)
                res = jnp.where(drop, jnp.float32(0.0), a)
                plsc.store_scatter(data, [idx], res)

        pltpu.sync_copy(data, o_hbm.at[pl.ds(r0 * F, BATCH_ROWS * F)])
        return carry

    lax.fori_loop(0, nbatch, batch_body, 0)


def _sc_kernel(x, mask_words):
    nrows = x.shape[0]
    rows_per_w = nrows // NW
    nbatch = rows_per_w // BATCH_ROWS
    mesh = plsc.VectorSubcoreMesh(core_axis_name="c", subcore_axis_name="s")
    k = pl.kernel(
        functools.partial(_sc_body, rows_per_w=rows_per_w, nbatch=nbatch),
        out_type=jax.ShapeDtypeStruct((nrows * F,), jnp.float32),
        mesh=mesh,
        scratch_types=[
            pltpu.VMEM((BATCH_ROWS * F,), jnp.float32),
            pltpu.VMEM((2048 * L,), jnp.int32),
            pltpu.VMEM((BATCH_ROWS * MASK_WORDS,), jnp.int32),
        ],
        compiler_params=pltpu.CompilerParams(needs_layout_passes=False),
    )
    flat = k(x.reshape(nrows * F), mask_words.reshape(nrows * MASK_WORDS))
    return flat.reshape(nrows, F)


def _count_lt(arr, cand, min_width):
    s = jnp.where(arr < cand, jnp.int16(1), jnp.int16(0))
    n = s.shape[1]
    while n > min_width:
        h = n // 2
        s = s[:, :h] + s[:, h:]
        n = h
    return jnp.sum(s.astype(jnp.int32), axis=1, keepdims=True)


def _tc16_body(x_ref, m_ref, o_ref, *, rank):
    x = x_ref[...]
    a = jnp.abs(x)
    bits = jax.lax.bitcast_convert_type(a, jnp.int32)
    rows = x.shape[0]
    hi = lax.shift_right_logical(bits, 16).astype(jnp.int16)

    def step1(_, carry):
        t, bit = carry
        cand = t | bit
        cnt = _count_lt(hi, cand.astype(jnp.int16), 256)
        t = jnp.where(cnt <= rank, cand, t)
        return t, bit >> 1

    t0 = jnp.zeros((rows, 1), dtype=jnp.int32)
    hi_t, _ = jax.lax.fori_loop(0, 15, step1, (t0, jnp.int32(1 << 14)))
    hi_t16 = hi_t.astype(jnp.int16)

    lo_s = ((bits & jnp.int32(0xFFFF)) - jnp.int32(32768)).astype(jnp.int16)
    below = hi < hi_t16
    eq = hi == hi_t16
    s2 = jnp.where(eq, lo_s, jnp.where(below, jnp.int16(-32768), jnp.int16(32767)))

    def step2(_, carry):
        t, bit = carry
        cand = t | bit
        cnt = _count_lt(s2, (cand - jnp.int32(32768)).astype(jnp.int16), 256)
        t = jnp.where(cnt <= rank, cand, t)
        return t, bit >> 1

    lo_t, _ = jax.lax.fori_loop(0, 16, step2, (t0, jnp.int32(1 << 15)))

    t_bits = (hi_t << 16) | lo_t
    thr = jax.lax.bitcast_convert_type(t_bits, jnp.float32)
    drop = (a * m_ref[...]) <= thr
    o_ref[...] = jnp.where(drop, jnp.zeros_like(a), a)


def _tc16_kernel(x, mask_scale_f32):
    nrows = x.shape[0]
    grid = (nrows // _ROW_BLOCK,)
    return pl.pallas_call(
        functools.partial(_tc16_body, rank=RANK),
        grid=grid,
        in_specs=[
            pl.BlockSpec((_ROW_BLOCK, F), lambda i: (i, 0)),
            pl.BlockSpec((_ROW_BLOCK, F), lambda i: (i, 0)),
        ],
        out_specs=pl.BlockSpec((_ROW_BLOCK, F), lambda i: (i, 0)),
        out_shape=jax.ShapeDtypeStruct((nrows, F), jnp.float32),
    )(x, mask_scale_f32)


@functools.cache
def _mask_scale_np():
    return np.where(_drop_mask_np(), np.float32(1.0),
                    np.float32(np.inf)).astype(np.float32)


def kernel(input):
    return _tc16_kernel(input, jnp.asarray(_mask_scale_np()))
